# baseline (speedup 1.0000x reference)
import functools

import jax
import jax.numpy as jnp
from jax import lax
from jax.experimental import pallas as pl
from jax.experimental.pallas import tpu as pltpu
from jax.experimental.pallas import tpu_sc as plsc

NUM_TRICKS = 100000
NUM_DRAFT_TRICKS = 1000
NUM_EMBEDDINGS = NUM_TRICKS + NUM_DRAFT_TRICKS
DRAFT_DELTA = NUM_TRICKS
DRAFT_PHASE = 2
EMBED_DIM = 128

NUM_WORKERS = 32
LANES = 16
CHUNK = 128
NBUF = 5


@functools.lru_cache(maxsize=None)
def _build(n_total):
    n = n_total // NUM_WORKERS
    n_units = n // CHUNK
    rounds = n_units // NBUF
    vec_per_unit = CHUNK // LANES
    mesh = plsc.VectorSubcoreMesh(core_axis_name="c", subcore_axis_name="s")

    @functools.partial(
        pl.kernel,
        mesh=mesh,
        compiler_params=pltpu.CompilerParams(use_tc_tiling_on_sc=False),
        out_type=jax.ShapeDtypeStruct((n_total, EMBED_DIM), jnp.float32),
        scratch_types=[
            pltpu.VMEM((n,), jnp.int32),
            pltpu.VMEM((n,), jnp.int32),
        ]
        + [pltpu.VMEM((CHUNK, EMBED_DIM), jnp.float32) for _ in range(NBUF)]
        + [pltpu.SemaphoreType.DMA for _ in range(2 * NBUF)],
    )
    def kern(trick_hbm, phase_hbm, table_hbm, out_hbm, idx_v, phase_v, *bufs_sems):
        rows = bufs_sems[:NBUF]
        g_sem = bufs_sems[NBUF:2 * NBUF]
        s_sem = bufs_sems[2 * NBUF:]
        wid = lax.axis_index("s") * 2 + lax.axis_index("c")
        base = wid * n
        pltpu.sync_copy(trick_hbm.at[pl.ds(base, n)], idx_v)
        pltpu.sync_copy(phase_hbm.at[pl.ds(base, n)], phase_v)

        def compute_idx(u):
            for j in range(vec_per_unit):
                o = u * CHUNK + j * LANES
                t = idx_v[pl.ds(o, LANES)]
                p = phase_v[pl.ds(o, LANES)]
                t = t + jnp.where(p == DRAFT_PHASE, DRAFT_DELTA, 0)
                idx_v[pl.ds(o, LANES)] = jnp.clip(t, -1, NUM_EMBEDDINGS - 1) + 1

        def gather_start(u, b):
            pltpu.async_copy(
                table_hbm.at[idx_v.at[pl.ds(u * CHUNK, CHUNK)]], rows[b], g_sem[b]
            )

        def gather_wait(u, b):
            pltpu.make_async_copy(
                table_hbm.at[idx_v.at[pl.ds(u * CHUNK, CHUNK)]], rows[b], g_sem[b]
            ).wait()

        def scatter_start(u, b):
            return pltpu.async_copy(
                rows[b], out_hbm.at[pl.ds(base + u * CHUNK, CHUNK)], s_sem[b]
            )

        for b in range(NBUF):
            compute_idx(b)
            gather_start(b, b)

        def main_round(r, carry):
            u0 = r * NBUF
            for b in range(NBUF):
                u = u0 + b
                gather_wait(u, b)
                sc = scatter_start(u, b)
                compute_idx(u + NBUF)
                sc.wait()
                gather_start(u + NBUF, b)
            return carry

        lax.fori_loop(0, rounds - 1, main_round, 0)

        u0 = (rounds - 1) * NBUF
        for b in range(NBUF):
            u = u0 + b
            gather_wait(u, b)
            scatter_start(u, b).wait()

    return kern


def kernel(trick, phase, table):
    b, h = trick.shape
    n_total = b * h
    out = _build(n_total)(
        trick.reshape(n_total).astype(jnp.int32),
        phase.reshape(n_total).astype(jnp.int32),
        table,
    )
    return out.reshape(b, h, EMBED_DIM)
